# Initial kernel scaffold; baseline (speedup 1.0000x reference)
import jax
import jax.numpy as jnp
from jax.experimental import pallas as pl


def kernel(x, lut):
    raise NotImplementedError("write your pallas kernel here")


# speedup vs baseline: 2.4135x; 2.4135x over previous
import functools
import math

import jax
import jax.numpy as jnp
from jax import lax
from jax.experimental import pallas as pl
from jax.experimental.pallas import tpu as pltpu
from jax.experimental.pallas import tpu_sc as plsc

D_MODEL = 128
SCALE = math.sqrt(float(D_MODEL))
NW = 32
C = 128
NCHUNK = 50
PER_W = C * NCHUNK
B = NW * PER_W
LANES = 16


def _body(x_hbm, lut_hbm, out_hbm, idx_v, rows_v, sem_g):
    wid = lax.axis_index("s") * 2 + lax.axis_index("c")
    base = wid * PER_W
    pltpu.sync_copy(x_hbm.at[wid], idx_v)

    def chunk_body(g, carry):
        pltpu.async_copy(lut_hbm.at[idx_v.at[g]], rows_v, sem_g).wait()

        def row_body(r, c2):
            for c8 in range(D_MODEL // LANES):
                sl = pl.ds(c8 * LANES, LANES)
                rows_v[r, sl] = rows_v[r, sl] * SCALE
            return c2

        lax.fori_loop(0, C, row_body, 0)
        pltpu.sync_copy(rows_v, out_hbm.at[pl.ds(base + g * C, C)])
        return carry

    lax.fori_loop(0, NCHUNK, chunk_body, 0)


_lookup = functools.partial(
    pl.kernel,
    out_type=jax.ShapeDtypeStruct((B, D_MODEL), jnp.float32),
    scratch_types=[
        pltpu.VMEM((NCHUNK, C), jnp.int32),
        pltpu.VMEM((C, D_MODEL), jnp.float32),
        pltpu.SemaphoreType.DMA,
    ],
    mesh=plsc.VectorSubcoreMesh(core_axis_name="c", subcore_axis_name="s"),
)(_body)


def kernel(x, lut):
    x_r = x.reshape(NW, NCHUNK, C)
    out = _lookup(x_r, lut)
    return out.reshape(x.shape[0], x.shape[1], D_MODEL)


# speedup vs baseline: 2.9446x; 1.2200x over previous
import functools
import math

import jax
import jax.numpy as jnp
from jax import lax
from jax.experimental import pallas as pl
from jax.experimental.pallas import tpu as pltpu
from jax.experimental.pallas import tpu_sc as plsc

D_MODEL = 128
SCALE = math.sqrt(float(D_MODEL))
NW = 32
C = 128
NCHUNK = 50
PER_W = C * NCHUNK
B = NW * PER_W
LANES = 16


def _body(x_hbm, lut_hbm, out_hbm, idx_v, in0, in1, out0, out1, sg0, sg1,
          ss0, ss1):
    wid = lax.axis_index("s") * 2 + lax.axis_index("c")
    base = wid * PER_W
    pltpu.sync_copy(x_hbm.at[wid], idx_v)

    ins = (in0, in1)
    outs = (out0, out1)
    sgs = (sg0, sg1)
    sss = (ss0, ss1)

    def scale_chunk(src, dst):
        def row_body(r, carry):
            for c8 in range(D_MODEL // LANES):
                sl = pl.ds(c8 * LANES, LANES)
                dst[r, sl] = src[r, sl] * SCALE
            return carry

        lax.fori_loop(0, C, row_body, 0)

    def pipe_step(c, b, first, last):
        inb, outb, sgb, ssb = ins[b], outs[b], sgs[b], sss[b]
        pltpu.make_async_copy(lut_hbm.at[idx_v.at[c]], inb, sgb).wait()
        if not first:
            pltpu.make_async_copy(
                outb, out_hbm.at[pl.ds(base + (c - 2) * C, C)], ssb).wait()
        scale_chunk(inb, outb)
        pltpu.async_copy(outb, out_hbm.at[pl.ds(base + c * C, C)], ssb)
        if not last:
            pltpu.async_copy(lut_hbm.at[idx_v.at[c + 2]], inb, sgb)

    pltpu.async_copy(lut_hbm.at[idx_v.at[0]], in0, sg0)
    pltpu.async_copy(lut_hbm.at[idx_v.at[1]], in1, sg1)
    pipe_step(0, 0, first=True, last=False)
    pipe_step(1, 1, first=True, last=False)

    def loop_body(i, carry):
        for b in range(2):
            pipe_step(2 * i + b, b, first=False, last=False)
        return carry

    lax.fori_loop(1, NCHUNK // 2 - 1, loop_body, 0)

    pipe_step(NCHUNK - 2, 0, first=False, last=True)
    pipe_step(NCHUNK - 1, 1, first=False, last=True)
    pltpu.make_async_copy(
        out0, out_hbm.at[pl.ds(base + (NCHUNK - 2) * C, C)], ss0).wait()
    pltpu.make_async_copy(
        out1, out_hbm.at[pl.ds(base + (NCHUNK - 1) * C, C)], ss1).wait()


_lookup = functools.partial(
    pl.kernel,
    out_type=jax.ShapeDtypeStruct((B, D_MODEL), jnp.float32),
    scratch_types=[
        pltpu.VMEM((NCHUNK, C), jnp.int32),
        pltpu.VMEM((C, D_MODEL), jnp.float32),
        pltpu.VMEM((C, D_MODEL), jnp.float32),
        pltpu.VMEM((C, D_MODEL), jnp.float32),
        pltpu.VMEM((C, D_MODEL), jnp.float32),
        pltpu.SemaphoreType.DMA,
        pltpu.SemaphoreType.DMA,
        pltpu.SemaphoreType.DMA,
        pltpu.SemaphoreType.DMA,
    ],
    mesh=plsc.VectorSubcoreMesh(core_axis_name="c", subcore_axis_name="s"),
)(_body)


def kernel(x, lut):
    x_r = x.reshape(NW, NCHUNK, C)
    out = _lookup(x_r, lut)
    return out.reshape(x.shape[0], x.shape[1], D_MODEL)


# speedup vs baseline: 5.2207x; 1.7730x over previous
import functools
import math

import jax
import jax.numpy as jnp
from jax import lax
from jax.experimental import pallas as pl
from jax.experimental.pallas import tpu as pltpu
from jax.experimental.pallas import tpu_sc as plsc

D_MODEL = 128
SCALE = math.sqrt(float(D_MODEL))
NW = 32
NROW = 4096
NCOL = 50
ROWS_PER_W = NROW // NW
R = 4
NCHUNK = ROWS_PER_W // R
LANES = 16


def _body(x_hbm, lut_hbm, out_hbm, idx_v, in0, in1, out0, out1, sg0, sg1,
          ss0, ss1):
    wid = lax.axis_index("s") * 2 + lax.axis_index("c")
    row0 = wid * ROWS_PER_W
    pltpu.sync_copy(x_hbm.at[pl.ds(row0, ROWS_PER_W)], idx_v)

    ins = (in0, in1)
    outs = (out0, out1)
    sgs = (sg0, sg1)
    sss = (ss0, ss1)

    def start_gathers(c, b):
        for j in range(R):
            pltpu.async_copy(lut_hbm.at[idx_v.at[c * R + j]], ins[b].at[j],
                             sgs[b])

    def wait_gathers(c, b):
        for j in range(R):
            pltpu.make_async_copy(lut_hbm.at[idx_v.at[c * R + j]],
                                  ins[b].at[j], sgs[b]).wait()

    def scale_chunk(src, dst):
        def row_body(r, carry):
            for j in range(R):
                for c8 in range(D_MODEL // LANES):
                    sl = pl.ds(c8 * LANES, LANES)
                    dst[j, r, sl] = src[j, r, sl] * SCALE
            return carry

        lax.fori_loop(0, NCOL, row_body, 0)

    def out_slice(c):
        return out_hbm.at[pl.ds(row0 + c * R, R)]

    def pipe_step(c, b, first, last):
        wait_gathers(c, b)
        if not first:
            pltpu.make_async_copy(outs[b], out_slice(c - 2), sss[b]).wait()
        scale_chunk(ins[b], outs[b])
        pltpu.async_copy(outs[b], out_slice(c), sss[b])
        if not last:
            start_gathers(c + 2, b)

    start_gathers(0, 0)
    start_gathers(1, 1)
    pipe_step(0, 0, first=True, last=False)
    pipe_step(1, 1, first=True, last=False)

    def loop_body(i, carry):
        for b in range(2):
            pipe_step(2 * i + b, b, first=False, last=False)
        return carry

    lax.fori_loop(1, NCHUNK // 2 - 1, loop_body, 0)

    pipe_step(NCHUNK - 2, 0, first=False, last=True)
    pipe_step(NCHUNK - 1, 1, first=False, last=True)
    pltpu.make_async_copy(out0, out_slice(NCHUNK - 2), ss0).wait()
    pltpu.make_async_copy(out1, out_slice(NCHUNK - 1), ss1).wait()


_lookup = functools.partial(
    pl.kernel,
    out_type=jax.ShapeDtypeStruct((NROW, NCOL, D_MODEL), jnp.float32),
    scratch_types=[
        pltpu.VMEM((ROWS_PER_W, NCOL), jnp.int32),
        pltpu.VMEM((R, NCOL, D_MODEL), jnp.float32),
        pltpu.VMEM((R, NCOL, D_MODEL), jnp.float32),
        pltpu.VMEM((R, NCOL, D_MODEL), jnp.float32),
        pltpu.VMEM((R, NCOL, D_MODEL), jnp.float32),
        pltpu.SemaphoreType.DMA,
        pltpu.SemaphoreType.DMA,
        pltpu.SemaphoreType.DMA,
        pltpu.SemaphoreType.DMA,
    ],
    mesh=plsc.VectorSubcoreMesh(core_axis_name="c", subcore_axis_name="s"),
    compiler_params=pltpu.CompilerParams(use_tc_tiling_on_sc=True),
)(_body)


def kernel(x, lut):
    return _lookup(x, lut)


# speedup vs baseline: 9.1523x; 1.7531x over previous
import functools
import math

import jax
import jax.numpy as jnp
from jax import lax
from jax.experimental import pallas as pl
from jax.experimental.pallas import tpu as pltpu
from jax.experimental.pallas import tpu_sc as plsc

D_MODEL = 128
SCALE = math.sqrt(float(D_MODEL))
NW = 32
NROW = 4096
NCOL = 50
C = NROW // NW
LANES = 16


def _body(xt_hbm, lut_hbm, out_hbm, idx_v, in0, in1, out0, out1, sg0, sg1,
          ss0, ss1):
    wid = lax.axis_index("s") * 2 + lax.axis_index("c")
    i0 = wid * C
    pltpu.sync_copy(xt_hbm.at[:, pl.ds(i0, C)], idx_v)

    ins = (in0, in1)
    outs = (out0, out1)
    sgs = (sg0, sg1)
    sss = (ss0, ss1)

    def scale_chunk(src, dst):
        def row_body(r, carry):
            for c8 in range(D_MODEL // LANES):
                sl = pl.ds(c8 * LANES, LANES)
                dst[r, sl] = src[r, sl] * SCALE
            return carry

        lax.fori_loop(0, C, row_body, 0)

    def pipe_step(c, b, first, last):
        pltpu.make_async_copy(lut_hbm.at[idx_v.at[c]], ins[b], sgs[b]).wait()
        if not first:
            pltpu.make_async_copy(
                outs[b], out_hbm.at[c - 2, pl.ds(i0, C)], sss[b]).wait()
        scale_chunk(ins[b], outs[b])
        pltpu.async_copy(outs[b], out_hbm.at[c, pl.ds(i0, C)], sss[b])
        if not last:
            pltpu.async_copy(lut_hbm.at[idx_v.at[c + 2]], ins[b], sgs[b])

    pltpu.async_copy(lut_hbm.at[idx_v.at[0]], in0, sg0)
    pltpu.async_copy(lut_hbm.at[idx_v.at[1]], in1, sg1)
    pipe_step(0, 0, first=True, last=False)
    pipe_step(1, 1, first=True, last=False)

    def loop_body(i, carry):
        for b in range(2):
            pipe_step(2 * i + b, b, first=False, last=False)
        return carry

    lax.fori_loop(1, NCOL // 2 - 1, loop_body, 0)

    pipe_step(NCOL - 2, 0, first=False, last=True)
    pipe_step(NCOL - 1, 1, first=False, last=True)
    pltpu.make_async_copy(
        out0, out_hbm.at[NCOL - 2, pl.ds(i0, C)], ss0).wait()
    pltpu.make_async_copy(
        out1, out_hbm.at[NCOL - 1, pl.ds(i0, C)], ss1).wait()


_lookup = functools.partial(
    pl.kernel,
    out_type=jax.ShapeDtypeStruct((NCOL, NROW, D_MODEL), jnp.float32),
    scratch_types=[
        pltpu.VMEM((NCOL, C), jnp.int32),
        pltpu.VMEM((C, D_MODEL), jnp.float32),
        pltpu.VMEM((C, D_MODEL), jnp.float32),
        pltpu.VMEM((C, D_MODEL), jnp.float32),
        pltpu.VMEM((C, D_MODEL), jnp.float32),
        pltpu.SemaphoreType.DMA,
        pltpu.SemaphoreType.DMA,
        pltpu.SemaphoreType.DMA,
        pltpu.SemaphoreType.DMA,
    ],
    mesh=plsc.VectorSubcoreMesh(core_axis_name="c", subcore_axis_name="s"),
    compiler_params=pltpu.CompilerParams(use_tc_tiling_on_sc=True),
)(_body)


def kernel(x, lut):
    out = _lookup(x.T, lut)
    return out.transpose(1, 0, 2)
